# baseline (speedup 1.0000x reference)
import functools

import jax
import jax.numpy as jnp
from jax import lax
from jax.experimental import pallas as pl
from jax.experimental.pallas import tpu as pltpu
from jax.experimental.pallas import tpu_sc as plsc

B = 16384
D = 64


def _gather_rows_sc(user_ids, item_ids, theta, beta):
    info = plsc.get_sparse_core_info()
    nc, ns = info.num_cores, info.num_subcores
    nw = nc * ns
    bpw = B // nw

    mesh = plsc.VectorSubcoreMesh(core_axis_name="c", subcore_axis_name="s")

    @functools.partial(
        pl.kernel,
        mesh=mesh,
        out_type=(
            jax.ShapeDtypeStruct((B, D), jnp.float32),
            jax.ShapeDtypeStruct((B, D), jnp.float32),
        ),
        scratch_types=[
            pltpu.VMEM((bpw,), jnp.int32),
            pltpu.VMEM((bpw,), jnp.int32),
            pltpu.VMEM((bpw, D), jnp.float32),
            pltpu.VMEM((bpw, D), jnp.float32),
            pltpu.SemaphoreType.DMA,
            pltpu.SemaphoreType.DMA,
        ],
        compiler_params=pltpu.CompilerParams(use_tc_tiling_on_sc=False),
    )
    def gather_kernel(uid_hbm, iid_hbm, theta_hbm, beta_hbm, out_t, out_b,
                      uidx_v, iidx_v, trows_v, brows_v, sem_t, sem_b):
        wid = lax.axis_index("s") * nc + lax.axis_index("c")
        base = wid * bpw
        pltpu.sync_copy(uid_hbm.at[pl.ds(base, bpw)], uidx_v)
        pltpu.sync_copy(iid_hbm.at[pl.ds(base, bpw)], iidx_v)
        cp_t = pltpu.async_copy(theta_hbm.at[uidx_v], trows_v, sem_t)
        cp_b = pltpu.async_copy(beta_hbm.at[iidx_v], brows_v, sem_b)
        cp_t.wait()
        pltpu.sync_copy(trows_v, out_t.at[pl.ds(base, bpw)])
        cp_b.wait()
        pltpu.sync_copy(brows_v, out_b.at[pl.ds(base, bpw)])

    return gather_kernel(user_ids, item_ids, theta, beta)


def _softplus(x):
    return jnp.maximum(x, 0.0) + jnp.log(1.0 + jnp.exp(-jnp.abs(x)))


def _dot_body(t_ref, b_ref, o_ref):
    sp_t = _softplus(t_ref[...])
    sp_b = _softplus(b_ref[...])
    o_ref[...] = jnp.sum(sp_t * sp_b, axis=1)


def _tc_reduce(trows, brows):
    blk = 2048
    return pl.pallas_call(
        _dot_body,
        grid=(B // blk,),
        in_specs=[
            pl.BlockSpec((blk, D), lambda i: (i, 0)),
            pl.BlockSpec((blk, D), lambda i: (i, 0)),
        ],
        out_specs=pl.BlockSpec((blk,), lambda i: (i,)),
        out_shape=jax.ShapeDtypeStruct((B,), jnp.float32),
    )(trows, brows)


def kernel(user_ids, item_ids, theta_uncons, beta_uncons):
    uid = user_ids.astype(jnp.int32)
    iid = item_ids.astype(jnp.int32)
    trows, brows = _gather_rows_sc(uid, iid, theta_uncons, beta_uncons)
    return _tc_reduce(trows, brows)
